# baseline (speedup 1.0000x reference)
import dataclasses
import functools

import jax
import jax.numpy as jnp
from jax import lax
from jax.experimental import pallas as pl
from jax.experimental.pallas import tpu as pltpu
from jax.experimental.pallas import tpu_sc as plsc

N_NODES = 10000
N_EDGES = 320000

NUM_CORES = 2
NUM_SUBCORES = 16
NUM_WORKERS = NUM_CORES * NUM_SUBCORES
LANES = 16

CHUNK = 128
ROWS_PAD = 2560
N_DUMMY = ROWS_PAD * CHUNK - N_EDGES
FEAT = 64

N_PAD = 10240
NODE_ROWS_PER_SUBCORE = N_PAD // NUM_SUBCORES
EDGE_ROWS_PER_TASK = ROWS_PAD // NUM_WORKERS
FEAT_ROWS_PER_TASK = ROWS_PAD // NUM_SUBCORES


_mesh = lambda: plsc.VectorSubcoreMesh(core_axis_name="c", subcore_axis_name="s")


def _sc_params():
    cp = pltpu.CompilerParams()
    if "needs_layout_passes" in pltpu.CompilerParams.__dataclass_fields__:
        cp = dataclasses.replace(cp, needs_layout_passes=False)
    cp = dataclasses.replace(cp, use_tc_tiling_on_sc=False)
    return cp


def _deg_partials(dst_rows):
    @functools.partial(
        pl.kernel,
        out_type=jax.ShapeDtypeStruct((NUM_WORKERS * N_NODES,), jnp.float32),
        mesh=_mesh(),
        compiler_params=_sc_params(),
        scratch_types=[
            pltpu.VMEM((EDGE_ROWS_PER_TASK, CHUNK), jnp.int32),
            pltpu.VMEM((N_PAD,), jnp.float32),
        ],
    )
    def deg_kernel(dst_hbm, out_hbm, idx_v, deg_v):
        c = lax.axis_index("c")
        s = lax.axis_index("s")
        wid = c * NUM_SUBCORES + s
        row0 = wid * EDGE_ROWS_PER_TASK
        pltpu.sync_copy(dst_hbm.at[pl.ds(row0, EDGE_ROWS_PER_TASK)], idx_v)

        zeros = jnp.zeros((LANES,), jnp.float32)

        @pl.loop(0, N_PAD // LANES)
        def _(k):
            deg_v[pl.ds(k * LANES, LANES)] = zeros

        ones = jnp.ones((LANES,), jnp.float32)

        @pl.loop(0, EDGE_ROWS_PER_TASK)
        def _(i):
            for j in range(CHUNK // LANES):
                idx = idx_v[i, pl.ds(j * LANES, LANES)]
                plsc.addupdate_scatter(deg_v, [idx], ones)

        pltpu.sync_copy(deg_v.at[pl.ds(0, N_NODES)],
                        out_hbm.at[pl.ds(wid * N_NODES, N_NODES)])

    return deg_kernel(dst_rows)


def _make_agg(mode):
    feat_mode = mode == "feat"
    rows_buf = FEAT_ROWS_PER_TASK if feat_mode else EDGE_ROWS_PER_TASK
    nrows = rows_buf

    @functools.partial(
        pl.kernel,
        out_type=jax.ShapeDtypeStruct((NUM_CORES * N_PAD, FEAT), jnp.float32),
        mesh=_mesh(),
        compiler_params=_sc_params(),
        scratch_types=[
            pltpu.VMEM((rows_buf, CHUNK), jnp.int32),
            pltpu.VMEM((rows_buf, CHUNK), jnp.int32),
            pltpu.VMEM((CHUNK, FEAT), jnp.float32),
            pltpu.VMEM((CHUNK, FEAT), jnp.float32),
            pltpu.VMEM((CHUNK, FEAT), jnp.float32),
            pltpu.VMEM((CHUNK, FEAT), jnp.float32),
            pltpu.VMEM_SHARED((N_PAD, FEAT), jnp.float32),
            pltpu.SemaphoreType.DMA,
            pltpu.SemaphoreType.DMA,
            pltpu.SemaphoreType.DMA,
            pltpu.SemaphoreType.DMA,
            pltpu.SemaphoreType.DMA,
            pltpu.SemaphoreType.DMA,
            pltpu.SemaphoreType.DMA,
            pltpu.SemaphoreType.DMA,
        ],
    )
    def agg_kernel(g_hbm, src_hbm, dst_hbm, out_hbm, src_v, dst_v,
                   b0, b1, b2, b3, acc,
                   g0, g1, g2, g3, s0, s1, s2, s3):
        bufs = (b0, b1, b2, b3)
        gsem = (g0, g1, g2, g3)
        ssem = (s0, s1, s2, s3)
        c = lax.axis_index("c")
        s = lax.axis_index("s")
        tid = s if feat_mode else c * NUM_SUBCORES + s
        row0 = tid * rows_buf
        pltpu.sync_copy(src_hbm.at[pl.ds(row0, rows_buf)], src_v)
        pltpu.sync_copy(dst_hbm.at[pl.ds(row0, rows_buf)], dst_v)

        if feat_mode:
            off = jnp.broadcast_to(c * N_NODES, (LANES,)).astype(jnp.int32)

            @pl.loop(0, rows_buf)
            def _(r):
                for jj in range(CHUNK // LANES):
                    sl = pl.ds(jj * LANES, LANES)
                    src_v[r, sl] = src_v[r, sl] + off

        zeros = jnp.zeros((LANES,), jnp.float32)

        @pl.loop(0, CHUNK)
        def _(r):
            for cc in range(FEAT // LANES):
                b0[r, pl.ds(cc * LANES, LANES)] = zeros

        base = s * NODE_ROWS_PER_SUBCORE
        for k in range(NODE_ROWS_PER_SUBCORE // CHUNK):
            pltpu.sync_copy(b0, acc.at[pl.ds(base + k * CHUNK, CHUNK)])
        plsc.subcore_barrier()

        def g_issue(i, k):
            pltpu.async_copy(g_hbm.at[src_v.at[i]], bufs[k], gsem[k])

        def g_wait(i, k):
            pltpu.make_async_copy(g_hbm.at[src_v.at[i]], bufs[k], gsem[k]).wait()

        def s_issue(i, k):
            pltpu.async_copy(bufs[k], acc.at[dst_v.at[i]], ssem[k], add=True)

        def s_wait(i, k):
            pltpu.make_async_copy(bufs[k], acc.at[dst_v.at[i]], ssem[k]).wait()

        g_issue(0, 0)
        g_issue(1, 1)
        for i in (0, 1):
            g_wait(i, i)
            s_issue(i, i)
            g_issue(i + 2, i + 2)

        @pl.loop(0, (nrows - 4) // 4)
        def _(j):
            ib = 2 + 4 * j
            for t in range(4):
                i = ib + t
                k = (2 + t) % 4
                g_wait(i, k)
                s_issue(i, k)
                s_wait(i - 2, t)
                g_issue(i + 2, t)

        for t, i in ((2, nrows - 2), (3, nrows - 1)):
            g_wait(i, t)
            s_issue(i, t)
        s_wait(nrows - 4, 0)
        s_wait(nrows - 3, 1)
        s_wait(nrows - 2, 2)
        s_wait(nrows - 1, 3)

        plsc.subcore_barrier()
        pltpu.sync_copy(
            acc.at[pl.ds(base, NODE_ROWS_PER_SUBCORE)],
            out_hbm.at[pl.ds(c * N_PAD + base, NODE_ROWS_PER_SUBCORE)],
        )

    return agg_kernel


_agg_feat = _make_agg("feat")
_agg_edge = _make_agg("edge")


_BR = 1000
_NB = N_NODES // _BR


def _dinv_body(parts_ref, o_ref):
    deg = 1.0 + jnp.sum(parts_ref[...], axis=0, keepdims=True)
    o_ref[...] = jnp.transpose(lax.rsqrt(deg))


def _dinv_col(parts):
    return pl.pallas_call(
        _dinv_body,
        out_shape=jax.ShapeDtypeStruct((N_NODES, 1), jnp.float32),
    )(parts)


def _mm1_scale_body(x_ref, w_ref, d_ref, o_ref):
    i = pl.program_id(0)
    h = jnp.dot(x_ref[...], w_ref[...], preferred_element_type=jnp.float32,
                precision=lax.Precision.HIGHEST) * d_ref[...]
    o_ref[...] = jnp.where(i < _NB, h[:, :FEAT], h[:, FEAT:])


def _mm1_scale(x, w1, dcol):
    n, kin = x.shape
    return pl.pallas_call(
        _mm1_scale_body,
        grid=(2 * _NB,),
        in_specs=[
            pl.BlockSpec((_BR, kin), lambda i: (i % _NB, 0)),
            pl.BlockSpec((kin, 2 * FEAT), lambda i: (0, 0)),
            pl.BlockSpec((_BR, 1), lambda i: (i % _NB, 0)),
        ],
        out_specs=pl.BlockSpec((_BR, FEAT), lambda i: (i, 0)),
        out_shape=jax.ShapeDtypeStruct((2 * n, FEAT), jnp.float32),
    )(x, w1, dcol)


def _mm2_body(p0_ref, p1_ref, ga_ref, gb_ref, d_ref, b_ref, w_ref, o_ref):
    d = d_ref[...]
    b = b_ref[...]
    w = w_ref[...]
    x1a = jnp.maximum(d * (p0_ref[...] + ga_ref[...]) + b[:, :FEAT], 0.0)
    x1b = jnp.maximum(d * (p1_ref[...] + gb_ref[...]) + b[:, FEAT:], 0.0)
    acc = jnp.dot(x1a, w[:FEAT, :], preferred_element_type=jnp.float32,
                  precision=lax.Precision.HIGHEST)
    acc += jnp.dot(x1b, w[FEAT:, :], preferred_element_type=jnp.float32,
                   precision=lax.Precision.HIGHEST)
    o_ref[...] = acc * d


def _mm2(p0, p1, ga, gb, dcol, brow, w2):
    n = p0.shape[0]
    kout = w2.shape[1]
    blk = pl.BlockSpec((_BR, FEAT), lambda i: (i, 0))
    return pl.pallas_call(
        _mm2_body,
        grid=(n // _BR,),
        in_specs=[
            blk, blk, blk, blk,
            pl.BlockSpec((_BR, 1), lambda i: (i, 0)),
            pl.BlockSpec((1, 2 * FEAT), lambda i: (0, 0)),
            pl.BlockSpec((2 * FEAT, kout), lambda i: (0, 0)),
        ],
        out_specs=pl.BlockSpec((_BR, kout), lambda i: (i, 0)),
        out_shape=jax.ShapeDtypeStruct((n, kout), jnp.float32),
    )(p0, p1, ga, gb, dcol, brow, w2)


def _final_body(q0_ref, q1_ref, g_ref, d_ref, b_ref, o_ref):
    z = d_ref[...] * (q0_ref[...] + q1_ref[...] + g_ref[...]) + b_ref[...]
    m = jnp.max(z, axis=-1, keepdims=True)
    e = z - m
    o_ref[...] = e - jnp.log(jnp.sum(jnp.exp(e), axis=-1, keepdims=True))


def _final(q0, q1, g2, dcol, brow):
    n, f = g2.shape
    blk = pl.BlockSpec((_BR, f), lambda i: (i, 0))
    return pl.pallas_call(
        _final_body,
        grid=(n // _BR,),
        in_specs=[
            blk, blk, blk,
            pl.BlockSpec((_BR, 1), lambda i: (i, 0)),
            pl.BlockSpec((1, f), lambda i: (0, 0)),
        ],
        out_specs=blk,
        out_shape=jax.ShapeDtypeStruct((n, f), jnp.float32),
    )(q0, q1, g2, dcol, brow)


def kernel(x, edge_index, W1, b1, W2, b2):
    n = x.shape[0]
    ei = edge_index.astype(jnp.int32)
    pad_iota = jnp.arange(N_DUMMY, dtype=jnp.int32)
    src_pad = pad_iota % N_NODES
    dst_pad = N_NODES + pad_iota % (N_PAD - N_NODES)
    src_rows = jnp.concatenate([ei[0], src_pad]).reshape(ROWS_PAD, CHUNK)
    dst_rows = jnp.concatenate([ei[1], dst_pad]).reshape(ROWS_PAD, CHUNK)

    deg_parts = _deg_partials(dst_rows).reshape(NUM_WORKERS, N_NODES)
    dinv_col = _dinv_col(deg_parts)
    tab1 = _mm1_scale(x, W1, dinv_col)

    p = _agg_feat(tab1, src_rows, dst_rows)
    g2 = _mm2(p[:n], p[N_PAD:N_PAD + n], tab1[:n], tab1[n:], dinv_col,
              b1.reshape(1, -1), W2)

    q = _agg_edge(g2, src_rows, dst_rows)
    out = _final(q[:n], q[N_PAD:N_PAD + n], g2, dinv_col, b2.reshape(1, -1))
    return out


# speedup vs baseline: 1.0951x; 1.0951x over previous
import dataclasses
import functools

import numpy as np

import jax
import jax.numpy as jnp
from jax import lax
from jax.experimental import pallas as pl
from jax.experimental.pallas import tpu as pltpu
from jax.experimental.pallas import tpu_sc as plsc

N_NODES = 10000
N_EDGES = 320000

NUM_CORES = 2
NUM_SUBCORES = 16
NUM_WORKERS = NUM_CORES * NUM_SUBCORES
LANES = 16

CHUNK = 128
ROWS_PAD = 2560
N_DUMMY = ROWS_PAD * CHUNK - N_EDGES
FEAT = 64

N_PAD = 10240
NODE_ROWS_PER_SUBCORE = N_PAD // NUM_SUBCORES
EDGE_ROWS_PER_TASK = ROWS_PAD // NUM_WORKERS
FEAT_ROWS_PER_TASK = ROWS_PAD // NUM_SUBCORES


_mesh = lambda: plsc.VectorSubcoreMesh(core_axis_name="c", subcore_axis_name="s")


def _sc_params():
    cp = pltpu.CompilerParams()
    if "needs_layout_passes" in pltpu.CompilerParams.__dataclass_fields__:
        cp = dataclasses.replace(cp, needs_layout_passes=False)
    cp = dataclasses.replace(cp, use_tc_tiling_on_sc=False)
    return cp


def _deg_partials(dst_rows):
    @functools.partial(
        pl.kernel,
        out_type=jax.ShapeDtypeStruct((NUM_WORKERS * N_NODES,), jnp.float32),
        mesh=_mesh(),
        compiler_params=_sc_params(),
        scratch_types=[
            pltpu.VMEM((EDGE_ROWS_PER_TASK, CHUNK), jnp.int32),
            pltpu.VMEM((N_PAD,), jnp.float32),
        ],
    )
    def deg_kernel(dst_hbm, out_hbm, idx_v, deg_v):
        c = lax.axis_index("c")
        s = lax.axis_index("s")
        wid = c * NUM_SUBCORES + s
        row0 = wid * EDGE_ROWS_PER_TASK
        pltpu.sync_copy(dst_hbm.at[pl.ds(row0, EDGE_ROWS_PER_TASK)], idx_v)

        zeros = jnp.zeros((LANES,), jnp.float32)

        @pl.loop(0, N_PAD // LANES)
        def _(k):
            deg_v[pl.ds(k * LANES, LANES)] = zeros

        ones = jnp.ones((LANES,), jnp.float32)

        @pl.loop(0, EDGE_ROWS_PER_TASK)
        def _(i):
            for j in range(CHUNK // LANES):
                idx = idx_v[i, pl.ds(j * LANES, LANES)]
                plsc.addupdate_scatter(deg_v, [idx], ones)

        pltpu.sync_copy(deg_v.at[pl.ds(0, N_NODES)],
                        out_hbm.at[pl.ds(wid * N_NODES, N_NODES)])

    return deg_kernel(dst_rows)


def _make_agg(mode):
    feat_mode = mode == "feat"
    rows_buf = FEAT_ROWS_PER_TASK if feat_mode else EDGE_ROWS_PER_TASK
    nrows = rows_buf

    kern = functools.partial(
        pl.kernel,
        out_type=jax.ShapeDtypeStruct((NUM_CORES * N_PAD, FEAT), jnp.float32),
        mesh=_mesh(),
        compiler_params=_sc_params(),
        scratch_types=[
            pltpu.VMEM((rows_buf, CHUNK), jnp.int32),
            pltpu.VMEM((rows_buf, CHUNK), jnp.int32),
            pltpu.VMEM((CHUNK, FEAT), jnp.float32),
            pltpu.VMEM((CHUNK, FEAT), jnp.float32),
            pltpu.VMEM((CHUNK, FEAT), jnp.float32),
            pltpu.VMEM((CHUNK, FEAT), jnp.float32),
            pltpu.VMEM_SHARED((N_PAD, FEAT), jnp.float32),
            pltpu.SemaphoreType.DMA,
            pltpu.SemaphoreType.DMA,
            pltpu.SemaphoreType.DMA,
            pltpu.SemaphoreType.DMA,
            pltpu.SemaphoreType.DMA,
            pltpu.SemaphoreType.DMA,
            pltpu.SemaphoreType.DMA,
            pltpu.SemaphoreType.DMA,
        ],
    )

    def agg_body(tabs, src_hbm, dst_hbm, out_hbm, src_v, dst_v,
                 b0, b1, b2, b3, acc,
                 g0, g1, g2, g3, s0, s1, s2, s3):
        bufs = (b0, b1, b2, b3)
        gsem = (g0, g1, g2, g3)
        ssem = (s0, s1, s2, s3)
        c = lax.axis_index("c")
        s = lax.axis_index("s")
        tid = s if feat_mode else c * NUM_SUBCORES + s
        row0 = tid * rows_buf
        pltpu.sync_copy(src_hbm.at[pl.ds(row0, rows_buf)], src_v)
        pltpu.sync_copy(dst_hbm.at[pl.ds(row0, rows_buf)], dst_v)

        zeros = jnp.zeros((LANES,), jnp.float32)

        @pl.loop(0, CHUNK)
        def _(r):
            for cc in range(FEAT // LANES):
                b0[r, pl.ds(cc * LANES, LANES)] = zeros

        base = s * NODE_ROWS_PER_SUBCORE
        for k in range(NODE_ROWS_PER_SUBCORE // CHUNK):
            pltpu.sync_copy(b0, acc.at[pl.ds(base + k * CHUNK, CHUNK)])
        plsc.subcore_barrier()

        def pipeline(g_hbm):
            def g_issue(i, k):
                pltpu.async_copy(g_hbm.at[src_v.at[i]], bufs[k], gsem[k])

            def g_wait(i, k):
                pltpu.make_async_copy(
                    g_hbm.at[src_v.at[i]], bufs[k], gsem[k]).wait()

            def s_issue(i, k):
                pltpu.async_copy(bufs[k], acc.at[dst_v.at[i]], ssem[k],
                                 add=True)

            def s_wait(i, k):
                pltpu.make_async_copy(
                    bufs[k], acc.at[dst_v.at[i]], ssem[k]).wait()

            g_issue(0, 0)
            g_issue(1, 1)
            for i in (0, 1):
                g_wait(i, i)
                s_issue(i, i)
                g_issue(i + 2, i + 2)

            @pl.loop(0, (nrows - 4) // 4)
            def _(j):
                ib = 2 + 4 * j
                for t in range(4):
                    i = ib + t
                    k = (2 + t) % 4
                    g_wait(i, k)
                    s_issue(i, k)
                    s_wait(i - 2, t)
                    g_issue(i + 2, t)

            for t, i in ((2, nrows - 2), (3, nrows - 1)):
                g_wait(i, t)
                s_issue(i, t)
            s_wait(nrows - 4, 0)
            s_wait(nrows - 3, 1)
            s_wait(nrows - 2, 2)
            s_wait(nrows - 1, 3)

        if feat_mode:
            @pl.when(c == 0)
            def _():
                pipeline(tabs[0])

            @pl.when(c == 1)
            def _():
                pipeline(tabs[1])
        else:
            pipeline(tabs[0])

        plsc.subcore_barrier()
        pltpu.sync_copy(
            acc.at[pl.ds(base, NODE_ROWS_PER_SUBCORE)],
            out_hbm.at[pl.ds(c * N_PAD + base, NODE_ROWS_PER_SUBCORE)],
        )

    if feat_mode:
        @kern
        def agg_kernel(ga, gb, src_hbm, dst_hbm, out_hbm, *rest):
            agg_body((ga, gb), src_hbm, dst_hbm, out_hbm, *rest)
    else:
        @kern
        def agg_kernel(g, src_hbm, dst_hbm, out_hbm, *rest):
            agg_body((g,), src_hbm, dst_hbm, out_hbm, *rest)

    return agg_kernel


_agg_feat = _make_agg("feat")
_agg_edge = _make_agg("edge")


_BR = 1000
_NB = N_NODES // _BR


def _dinv_body(parts_ref, o_ref):
    deg = 1.0 + jnp.sum(parts_ref[...], axis=0, keepdims=True)
    o_ref[...] = jnp.transpose(lax.rsqrt(deg))


def _dinv_col(parts):
    return pl.pallas_call(
        _dinv_body,
        out_shape=jax.ShapeDtypeStruct((N_NODES, 1), jnp.float32),
    )(parts)


def _mm1_scale_body(x_ref, w_ref, d_ref, oa_ref, ob_ref):
    h = jnp.dot(x_ref[...], w_ref[...],
                preferred_element_type=jnp.float32) * d_ref[...]
    oa_ref[...] = h[:, :FEAT]
    ob_ref[...] = h[:, FEAT:]


def _mm1_scale(x, w1, dcol):
    n, kin = x.shape
    out = jax.ShapeDtypeStruct((n, FEAT), jnp.float32)
    return pl.pallas_call(
        _mm1_scale_body,
        grid=(_NB,),
        in_specs=[
            pl.BlockSpec((_BR, kin), lambda i: (i, 0)),
            pl.BlockSpec((kin, 2 * FEAT), lambda i: (0, 0)),
            pl.BlockSpec((_BR, 1), lambda i: (i, 0)),
        ],
        out_specs=[
            pl.BlockSpec((_BR, FEAT), lambda i: (i, 0)),
            pl.BlockSpec((_BR, FEAT), lambda i: (i, 0)),
        ],
        out_shape=[out, out],
    )(x, w1, dcol)


def _mm2_body(p0_ref, p1_ref, ga_ref, gb_ref, d_ref, b_ref, w_ref, o_ref):
    d = d_ref[...]
    b = b_ref[...]
    w = w_ref[...]
    x1a = jnp.maximum(d * (p0_ref[...] + ga_ref[...]) + b[:, :FEAT], 0.0)
    x1b = jnp.maximum(d * (p1_ref[...] + gb_ref[...]) + b[:, FEAT:], 0.0)
    acc = jnp.dot(x1a, w[:FEAT, :], preferred_element_type=jnp.float32)
    acc += jnp.dot(x1b, w[FEAT:, :], preferred_element_type=jnp.float32)
    o_ref[...] = acc * d


def _mm2(p0, p1, ga, gb, dcol, brow, w2):
    n = p0.shape[0]
    kout = w2.shape[1]
    blk = pl.BlockSpec((_BR, FEAT), lambda i: (i, 0))
    return pl.pallas_call(
        _mm2_body,
        grid=(n // _BR,),
        in_specs=[
            blk, blk, blk, blk,
            pl.BlockSpec((_BR, 1), lambda i: (i, 0)),
            pl.BlockSpec((1, 2 * FEAT), lambda i: (0, 0)),
            pl.BlockSpec((2 * FEAT, kout), lambda i: (0, 0)),
        ],
        out_specs=pl.BlockSpec((_BR, kout), lambda i: (i, 0)),
        out_shape=jax.ShapeDtypeStruct((n, kout), jnp.float32),
    )(p0, p1, ga, gb, dcol, brow, w2)


def _final_body(q0_ref, q1_ref, g_ref, d_ref, b_ref, o_ref):
    z = d_ref[...] * (q0_ref[...] + q1_ref[...] + g_ref[...]) + b_ref[...]
    m = jnp.max(z, axis=-1, keepdims=True)
    e = z - m
    o_ref[...] = e - jnp.log(jnp.sum(jnp.exp(e), axis=-1, keepdims=True))


def _final(q0, q1, g2, dcol, brow):
    n, f = g2.shape
    blk = pl.BlockSpec((_BR, f), lambda i: (i, 0))
    return pl.pallas_call(
        _final_body,
        grid=(n // _BR,),
        in_specs=[
            blk, blk, blk,
            pl.BlockSpec((_BR, 1), lambda i: (i, 0)),
            pl.BlockSpec((1, f), lambda i: (0, 0)),
        ],
        out_specs=blk,
        out_shape=jax.ShapeDtypeStruct((n, f), jnp.float32),
    )(q0, q1, g2, dcol, brow)


def kernel(x, edge_index, W1, b1, W2, b2):
    n = x.shape[0]
    ei = edge_index.astype(jnp.int32)
    pad_np = np.arange(N_DUMMY, dtype=np.int32)
    src_pad = jnp.asarray(pad_np % N_NODES)
    dst_pad = jnp.asarray(N_NODES + pad_np % (N_PAD - N_NODES))
    src_rows = jnp.concatenate([ei[0], src_pad]).reshape(ROWS_PAD, CHUNK)
    dst_rows = jnp.concatenate([ei[1], dst_pad]).reshape(ROWS_PAD, CHUNK)

    deg_parts = _deg_partials(dst_rows).reshape(NUM_WORKERS, N_NODES)
    dinv_col = _dinv_col(deg_parts)
    tab_a, tab_b = _mm1_scale(x, W1, dinv_col)

    p = _agg_feat(tab_a, tab_b, src_rows, dst_rows)
    g2 = _mm2(p[:n], p[N_PAD:N_PAD + n], tab_a, tab_b, dinv_col,
              b1.reshape(1, -1), W2)

    q = _agg_edge(g2, src_rows, dst_rows)
    out = _final(q[:n], q[N_PAD:N_PAD + n], g2, dinv_col, b2.reshape(1, -1))
    return out
